# Initial kernel scaffold; baseline (speedup 1.0000x reference)
import jax
import jax.numpy as jnp
from jax.experimental import pallas as pl


def kernel(x, month_table, year_table):
    raise NotImplementedError("write your pallas kernel here")


# speedup vs baseline: 2.9092x; 2.9092x over previous
import jax
import jax.numpy as jnp
from jax import lax
from jax.experimental import pallas as pl
from jax.experimental.pallas import tpu as pltpu
from jax.experimental.pallas import tpu_sc as plsc

NC, NS = 2, 16
NW = NC * NS
B = 4096 * 50
D = 128
NM, NY = 13, 20
RPW = B // NW
CHUNK = 128
NCHUNK = RPW // CHUNK


def _comb_body(m_ref, y_ref, o_ref):
    o_ref[...] = m_ref[...][:, None, :] + y_ref[...][None, :, :]


def _build_comb(month_table, year_table):
    out = pl.pallas_call(
        _comb_body,
        out_shape=jax.ShapeDtypeStruct((NM, NY, D), jnp.float32),
    )(month_table, year_table)
    return out.reshape(NM * NY, D)


def _sc_body(x_hbm, comb_hbm, out_hbm, xv, idxb,
             rows0, rows1, gsem0, gsem1, ssem0, ssem1):
    wid = lax.axis_index("s") * NC + lax.axis_index("c")
    base = wid * RPW

    pltpu.sync_copy(x_hbm.at[pl.ds(base * 2, RPW * 2)], xv)

    def idx_chunk(c, carry):
        for j in range(CHUNK // 16):
            pos = (c * CHUNK + j * 16 + lax.iota(jnp.int32, 16)) * 2
            m = plsc.load_gather(xv, [pos])
            y = plsc.load_gather(xv, [pos + 1])
            idxb[c, pl.ds(j * 16, 16)] = m * NY + y
        return carry
    lax.fori_loop(0, NCHUNK, idx_chunk, 0)

    def pair(t, carry):
        c0 = t * 2
        c1 = c0 + 1
        g0 = pltpu.async_copy(comb_hbm.at[idxb.at[c0]], rows0, gsem0)
        g1 = pltpu.async_copy(comb_hbm.at[idxb.at[c1]], rows1, gsem1)
        g0.wait()
        s0 = pltpu.async_copy(
            rows0, out_hbm.at[pl.ds(base + c0 * CHUNK, CHUNK)], ssem0)
        g1.wait()
        s1 = pltpu.async_copy(
            rows1, out_hbm.at[pl.ds(base + c1 * CHUNK, CHUNK)], ssem1)
        s0.wait()
        s1.wait()
        return carry
    lax.fori_loop(0, NCHUNK // 2, pair, 0)


def _sc_call(x_flat, comb):
    mesh = plsc.VectorSubcoreMesh(
        core_axis_name="c", subcore_axis_name="s",
        num_cores=NC, num_subcores=NS)
    fn = pl.kernel(
        _sc_body,
        out_type=jax.ShapeDtypeStruct((B, D), jnp.float32),
        mesh=mesh,
        compiler_params=pltpu.CompilerParams(needs_layout_passes=False),
        scratch_types=[
            pltpu.VMEM((RPW * 2,), jnp.int32),
            pltpu.VMEM((NCHUNK, CHUNK), jnp.int32),
            pltpu.VMEM((CHUNK, D), jnp.float32),
            pltpu.VMEM((CHUNK, D), jnp.float32),
            pltpu.SemaphoreType.DMA,
            pltpu.SemaphoreType.DMA,
            pltpu.SemaphoreType.DMA,
            pltpu.SemaphoreType.DMA,
        ],
    )
    return fn(x_flat, comb)


def kernel(x, month_table, year_table):
    comb = _build_comb(month_table, year_table)
    x_flat = x.astype(jnp.int32).reshape(-1)
    out = _sc_call(x_flat, comb)
    return out.reshape(x.shape[0], x.shape[1], D)


# speedup vs baseline: 3.5498x; 1.2202x over previous
import jax
import jax.numpy as jnp
from jax import lax
from jax.experimental import pallas as pl
from jax.experimental.pallas import tpu as pltpu
from jax.experimental.pallas import tpu_sc as plsc

NC, NS = 2, 16
NW = NC * NS
NB = 4096
T = 50
D = 128
NM, NY = 13, 20
BPW = NB // NW
TP = 64
GROUP = 4
NGRP = BPW // GROUP


def _comb_body(m_ref, y_ref, o_ref):
    o_ref[...] = m_ref[...][:, None, :] + y_ref[...][None, :, :]


def _build_comb(month_table, year_table):
    out = pl.pallas_call(
        _comb_body,
        out_shape=jax.ShapeDtypeStruct((NM, NY, D), jnp.float32),
    )(month_table, year_table)
    return out.reshape(NM * NY, D)


def _sc_body(x_hbm, comb_hbm, out_hbm, xv, idxb, bank0, bank1,
             gsem0, gsem1, ssem0, ssem1):
    wid = lax.axis_index("s") * NC + lax.axis_index("c")
    b0 = wid * BPW

    pltpu.sync_copy(x_hbm.at[pl.ds(b0 * T * 2, BPW * T * 2)], xv)

    def idx_batch(b, carry):
        for j in range(TP // 16):
            t = jnp.minimum(j * 16 + lax.iota(jnp.int32, 16), T - 1)
            pos = (b * T + t) * 2
            m = plsc.load_gather(xv, [pos])
            y = plsc.load_gather(xv, [pos + 1])
            idxb[b, pl.ds(j * 16, 16)] = m * NY + y
        return carry
    lax.fori_loop(0, BPW, idx_batch, 0)

    def ig(bank, sem, g):
        for q in range(GROUP):
            pltpu.async_copy(
                comb_hbm.at[idxb.at[g * GROUP + q, pl.ds(0, T)]],
                bank.at[q], sem)

    def wg(bank, sem):
        for q in range(GROUP):
            pltpu.make_async_copy(
                comb_hbm.at[idxb.at[0, pl.ds(0, T)]], bank.at[q], sem).wait()

    def isc(bank, sem, g):
        for q in range(GROUP):
            pltpu.async_copy(bank.at[q], out_hbm.at[b0 + g * GROUP + q], sem)

    def wsc(bank, sem):
        for q in range(GROUP):
            pltpu.make_async_copy(bank.at[q], out_hbm.at[b0], sem).wait()

    ig(bank0, gsem0, 0)
    ig(bank1, gsem1, 1)

    def step(g, bank, gsem, ssem, prefetch):
        wg(bank, gsem)
        isc(bank, ssem, g)
        wsc(bank, ssem)
        if prefetch:
            ig(bank, gsem, g + 2)

    def pair(t, carry):
        step(t * 2, bank0, gsem0, ssem0, True)
        step(t * 2 + 1, bank1, gsem1, ssem1, True)
        return carry
    lax.fori_loop(0, NGRP // 2 - 1, pair, 0)

    step(NGRP - 2, bank0, gsem0, ssem0, False)
    step(NGRP - 1, bank1, gsem1, ssem1, False)


def _sc_call(x, comb):
    mesh = plsc.VectorSubcoreMesh(
        core_axis_name="c", subcore_axis_name="s",
        num_cores=NC, num_subcores=NS)
    fn = pl.kernel(
        _sc_body,
        out_type=jax.ShapeDtypeStruct((NB, T, D), jnp.float32),
        mesh=mesh,
        compiler_params=pltpu.CompilerParams(needs_layout_passes=False),
        scratch_types=[
            pltpu.VMEM((BPW * T * 2,), jnp.int32),
            pltpu.VMEM((BPW, TP), jnp.int32),
            pltpu.VMEM((GROUP, T, D), jnp.float32),
            pltpu.VMEM((GROUP, T, D), jnp.float32),
            pltpu.SemaphoreType.DMA,
            pltpu.SemaphoreType.DMA,
            pltpu.SemaphoreType.DMA,
            pltpu.SemaphoreType.DMA,
        ],
    )
    return fn(x, comb)


def kernel(x, month_table, year_table):
    comb = _build_comb(month_table, year_table)
    out = _sc_call(x.astype(jnp.int32).reshape(-1), comb)
    return out


# speedup vs baseline: 3.7065x; 1.0441x over previous
import jax
import jax.numpy as jnp
from jax import lax
from jax.experimental import pallas as pl
from jax.experimental.pallas import tpu as pltpu
from jax.experimental.pallas import tpu_sc as plsc

NC, NS = 2, 16
NW = NC * NS
NB = 4096
T = 50
D = 128
NM, NY = 13, 20
BPW = NB // NW
GROUP = 4
NGRP = BPW // GROUP
BS = 128


def _comb_body(m_ref, y_ref, o_ref):
    o_ref[...] = m_ref[...][:, None, :] + y_ref[...][None, :, :]


def _build_comb(month_table, year_table):
    out = pl.pallas_call(
        _comb_body,
        out_shape=jax.ShapeDtypeStruct((NM, NY, D), jnp.float32),
    )(month_table, year_table)
    return out.reshape(NM * NY, D)


def _cidx_body(x_ref, o_ref):
    o_ref[...] = x_ref[:, :, 0] * NY + x_ref[:, :, 1]


def _build_cidx(x):
    return pl.pallas_call(
        _cidx_body,
        grid=(NB // BS,),
        in_specs=[pl.BlockSpec((BS, T, 2), lambda i: (i, 0, 0))],
        out_specs=pl.BlockSpec((BS, T), lambda i: (i, 0)),
        out_shape=jax.ShapeDtypeStruct((NB, T), jnp.int32),
    )(x)


def _sc_body(cidx_hbm, comb_hbm, out_hbm, cv, bank0, bank1,
             gsem0, gsem1, ssem0, ssem1):
    wid = lax.axis_index("s") * NC + lax.axis_index("c")
    b0 = wid * BPW

    pltpu.sync_copy(cidx_hbm.at[pl.ds(b0, BPW)], cv)

    def ig(bank, sem, g):
        for q in range(GROUP):
            pltpu.async_copy(
                comb_hbm.at[cv.at[g * GROUP + q, pl.ds(0, T)]],
                bank.at[q], sem)

    def wg(bank, sem):
        for q in range(GROUP):
            pltpu.make_async_copy(
                comb_hbm.at[cv.at[0, pl.ds(0, T)]], bank.at[q], sem).wait()

    def isc(bank, sem, g):
        for q in range(GROUP):
            pltpu.async_copy(bank.at[q], out_hbm.at[b0 + g * GROUP + q], sem)

    def wsc(bank, sem):
        for q in range(GROUP):
            pltpu.make_async_copy(bank.at[q], out_hbm.at[b0], sem).wait()

    ig(bank0, gsem0, 0)
    ig(bank1, gsem1, 1)

    def step(g, bank, gsem, ssem, prefetch):
        wg(bank, gsem)
        isc(bank, ssem, g)
        wsc(bank, ssem)
        if prefetch:
            ig(bank, gsem, g + 2)

    def pair(t, carry):
        step(t * 2, bank0, gsem0, ssem0, True)
        step(t * 2 + 1, bank1, gsem1, ssem1, True)
        return carry
    lax.fori_loop(0, NGRP // 2 - 1, pair, 0)

    step(NGRP - 2, bank0, gsem0, ssem0, False)
    step(NGRP - 1, bank1, gsem1, ssem1, False)


def _sc_call(cidx, comb):
    mesh = plsc.VectorSubcoreMesh(
        core_axis_name="c", subcore_axis_name="s",
        num_cores=NC, num_subcores=NS)
    fn = pl.kernel(
        _sc_body,
        out_type=jax.ShapeDtypeStruct((NB, T, D), jnp.float32),
        mesh=mesh,
        compiler_params=pltpu.CompilerParams(
            needs_layout_passes=False, use_tc_tiling_on_sc=True),
        scratch_types=[
            pltpu.VMEM((BPW, T), jnp.int32),
            pltpu.VMEM((GROUP, T, D), jnp.float32),
            pltpu.VMEM((GROUP, T, D), jnp.float32),
            pltpu.SemaphoreType.DMA,
            pltpu.SemaphoreType.DMA,
            pltpu.SemaphoreType.DMA,
            pltpu.SemaphoreType.DMA,
        ],
    )
    return fn(cidx, comb)


def kernel(x, month_table, year_table):
    xi = x.astype(jnp.int32)
    comb = _build_comb(month_table, year_table)
    cidx = _build_cidx(xi)
    return _sc_call(cidx, comb)


# speedup vs baseline: 4.9620x; 1.3387x over previous
import jax
import jax.numpy as jnp
from jax import lax
from jax.experimental import pallas as pl
from jax.experimental.pallas import tpu as pltpu
from jax.experimental.pallas import tpu_sc as plsc

NC, NS = 2, 16
NW = NC * NS
NB = 4096
T = 50
D = 128
NM, NY = 13, 20
BPW = NB // NW
GROUP = 4
NGRP = BPW // GROUP
BS = 128


def _comb_body(m_ref, y_ref, o_ref):
    o_ref[...] = m_ref[...][:, None, :] + y_ref[...][None, :, :]


def _build_comb(month_table, year_table):
    out = pl.pallas_call(
        _comb_body,
        out_shape=jax.ShapeDtypeStruct((NM, NY, D), jnp.float32),
    )(month_table, year_table)
    return out.reshape(NM * NY, D)


def _cidx_body(x_ref, k_ref, o_ref):
    c = jnp.dot(x_ref[...].astype(jnp.float32), k_ref[...],
                preferred_element_type=jnp.float32)
    o_ref[...] = c.astype(jnp.int32)


def _build_cidx(x2d):
    tt = lax.broadcasted_iota(jnp.int32, (2 * T, T), 0)
    tc = lax.broadcasted_iota(jnp.int32, (2 * T, T), 1)
    k = jnp.where(tt == 2 * tc, 20.0,
                  jnp.where(tt == 2 * tc + 1, 1.0, 0.0)).astype(jnp.float32)
    return pl.pallas_call(
        _cidx_body,
        grid=(NB // BS,),
        in_specs=[pl.BlockSpec((BS, 2 * T), lambda i: (i, 0)),
                  pl.BlockSpec((2 * T, T), lambda i: (0, 0))],
        out_specs=pl.BlockSpec((BS, T), lambda i: (i, 0)),
        out_shape=jax.ShapeDtypeStruct((NB, T), jnp.int32),
    )(x2d, k)


def _sc_body(cidx_hbm, comb_hbm, out_hbm, cv, bank0, bank1,
             gsem0, gsem1, ssem0, ssem1):
    wid = lax.axis_index("s") * NC + lax.axis_index("c")
    b0 = wid * BPW

    pltpu.sync_copy(cidx_hbm.at[pl.ds(b0, BPW)], cv)

    def ig(bank, sem, g):
        for q in range(GROUP):
            pltpu.async_copy(
                comb_hbm.at[cv.at[g * GROUP + q, pl.ds(0, T)]],
                bank.at[q], sem)

    def wg(bank, sem):
        for q in range(GROUP):
            pltpu.make_async_copy(
                comb_hbm.at[cv.at[0, pl.ds(0, T)]], bank.at[q], sem).wait()

    def isc(bank, sem, g):
        for q in range(GROUP):
            pltpu.async_copy(bank.at[q], out_hbm.at[b0 + g * GROUP + q], sem)

    def wsc(bank, sem):
        for q in range(GROUP):
            pltpu.make_async_copy(bank.at[q], out_hbm.at[b0], sem).wait()

    ig(bank0, gsem0, 0)
    ig(bank1, gsem1, 1)

    def step(g, bank, gsem, ssem, prefetch):
        wg(bank, gsem)
        isc(bank, ssem, g)
        wsc(bank, ssem)
        if prefetch:
            ig(bank, gsem, g + 2)

    def pair(t, carry):
        step(t * 2, bank0, gsem0, ssem0, True)
        step(t * 2 + 1, bank1, gsem1, ssem1, True)
        return carry
    lax.fori_loop(0, NGRP // 2 - 1, pair, 0)

    step(NGRP - 2, bank0, gsem0, ssem0, False)
    step(NGRP - 1, bank1, gsem1, ssem1, False)


def _sc_call(cidx, comb):
    mesh = plsc.VectorSubcoreMesh(
        core_axis_name="c", subcore_axis_name="s",
        num_cores=NC, num_subcores=NS)
    fn = pl.kernel(
        _sc_body,
        out_type=jax.ShapeDtypeStruct((NB, T, D), jnp.float32),
        mesh=mesh,
        compiler_params=pltpu.CompilerParams(
            needs_layout_passes=False, use_tc_tiling_on_sc=True),
        scratch_types=[
            pltpu.VMEM((BPW, T), jnp.int32),
            pltpu.VMEM((GROUP, T, D), jnp.float32),
            pltpu.VMEM((GROUP, T, D), jnp.float32),
            pltpu.SemaphoreType.DMA,
            pltpu.SemaphoreType.DMA,
            pltpu.SemaphoreType.DMA,
            pltpu.SemaphoreType.DMA,
        ],
    )
    return fn(cidx, comb)


def kernel(x, month_table, year_table):
    x2d = x.astype(jnp.int32).reshape(NB, 2 * T)
    comb = _build_comb(month_table, year_table)
    cidx = _build_cidx(x2d)
    return _sc_call(cidx, comb)
